# baseline (speedup 1.0000x reference)
import functools

import jax
import jax.numpy as jnp
from jax import lax
from jax.experimental import pallas as pl
from jax.experimental.pallas import tpu as pltpu
from jax.experimental.pallas import tpu_sc as plsc

M_TOTAL = 2097152
B_CLONE = 262144
SIZE = M_TOTAL - B_CLONE

LANES = 128
R_TOTAL = M_TOTAL // LANES
R_HEAD = SIZE // LANES
R_TAIL = B_CLONE // LANES

RBLK = 512
GRID = R_TOTAL // RBLK
N_HEAD = R_HEAD // RBLK
HBLK = 256
H_STEPS = R_HEAD // HBLK


def _cov_math(s0, s1, s2, q0, q1, q2, q3):
    n2 = q0 * q0 + q1 * q1 + q2 * q2 + q3 * q3
    inv = 1.0 / jnp.maximum(jnp.sqrt(n2), 1e-12)
    w, x, y, z = q0 * inv, q1 * inv, q2 * inv, q3 * inv

    e0 = jnp.exp(s0)
    e1 = jnp.exp(s1)
    e2 = jnp.exp(s2)

    m00 = (1.0 - 2.0 * (y * y + z * z)) * e0
    m01 = (2.0 * (x * y - w * z)) * e1
    m02 = (2.0 * (x * z + w * y)) * e2
    m10 = (2.0 * (x * y + w * z)) * e0
    m11 = (1.0 - 2.0 * (x * x + z * z)) * e1
    m12 = (2.0 * (y * z - w * x)) * e2
    m20 = (2.0 * (x * z - w * y)) * e0
    m21 = (2.0 * (y * z + w * x)) * e1
    m22 = (1.0 - 2.0 * (x * x + y * y)) * e2

    c00 = m00 * m00 + m01 * m01 + m02 * m02
    c01 = m00 * m10 + m01 * m11 + m02 * m12
    c02 = m00 * m20 + m01 * m21 + m02 * m22
    c11 = m10 * m10 + m11 * m11 + m12 * m12
    c12 = m10 * m20 + m11 * m21 + m12 * m22
    c22 = m20 * m20 + m21 * m21 + m22 * m22
    return c00, c01, c02, c11, c12, c22


def _store_planes(outr, c00, c01, c02, c11, c12, c22):
    outr[0] = c00
    outr[1] = c01
    outr[2] = c02
    outr[3] = c01
    outr[4] = c11
    outr[5] = c12
    outr[6] = c02
    outr[7] = c12
    outr[8] = c22


def _cov_head_body(s0r, s1r, s2r, rotr, outr):
    v = rotr[...].reshape(HBLK, 4, LANES)
    _store_planes(outr, *_cov_math(s0r[...], s1r[...], s2r[...],
                                   v[:, 0, :], v[:, 1, :],
                                   v[:, 2, :], v[:, 3, :]))


def _cov_tail_body(s0r, s1r, s2r, q0r, q1r, q2r, q3r, alias_r, outr):
    del alias_r
    _store_planes(outr, *_cov_math(s0r[...], s1r[...], s2r[...],
                                   q0r[...], q1r[...], q2r[...], q3r[...]))


def _sc_gather(tables, idxs):
    nt = len(tables)
    uniq = []
    idx_of = []
    for a in idxs:
        for j, u in enumerate(uniq):
            if u is a:
                idx_of.append(j)
                break
        else:
            idx_of.append(len(uniq))
            uniq.append(a)
    nu = len(uniq)

    info = plsc.get_sparse_core_info()
    nc, ns = info.num_cores, info.num_subcores
    nw = nc * ns
    b_per_w = B_CLONE // nw
    mesh = plsc.VectorSubcoreMesh(core_axis_name="c", subcore_axis_name="s")

    @functools.partial(
        pl.kernel,
        mesh=mesh,
        out_type=[jax.ShapeDtypeStruct((B_CLONE,), jnp.float32)
                  for _ in range(nt)],
        scratch_types=(
            [pltpu.VMEM((b_per_w,), jnp.int32) for _ in range(nu)]
            + [pltpu.VMEM((b_per_w,), jnp.float32) for _ in range(nt)]
            + [pltpu.SemaphoreType.DMA for _ in range(nt)]
        ),
    )
    def gather_k(*refs):
        tbls = refs[:nt]
        idx_hbm = refs[nt:nt + nu]
        outs = refs[nt + nu:2 * nt + nu]
        idx_v = refs[2 * nt + nu:2 * nt + 2 * nu]
        stages = refs[2 * nt + 2 * nu:3 * nt + 2 * nu]
        sems = refs[3 * nt + 2 * nu:]
        wid = lax.axis_index("s") * nc + lax.axis_index("c")
        base = wid * b_per_w

        for u in range(nu):
            pltpu.sync_copy(idx_hbm[u].at[pl.ds(base, b_per_w)], idx_v[u])
        copies = [
            pltpu.async_copy(tbls[t].at[idx_v[idx_of[t]]], stages[t], sems[t])
            for t in range(nt)
        ]
        for t in range(nt):
            copies[t].wait()
            pltpu.sync_copy(stages[t], outs[t].at[pl.ds(base, b_per_w)])

    return gather_k(*tables, *uniq)


def kernel(scale, rotation, idx):
    scomps = [scale[:, c] for c in range(3)]
    rot_i = rotation.reshape(R_TOTAL, LANES, 4).transpose(0, 2, 1)
    rot2 = rot_i.reshape(4 * R_TOTAL, LANES)
    rot_flat = rot_i.reshape(M_TOTAL * 4)

    rbase = (idx // LANES) * (4 * LANES) + (idx % LANES)
    ridx = [rbase + c * LANES for c in range(4)]

    tables = [c.reshape(M_TOTAL) for c in scomps] + [rot_flat] * 4
    gathered = _sc_gather(tables, [idx, idx, idx] + ridx)

    shead = [c.reshape(R_TOTAL, LANES) for c in scomps]
    tail_in = [g.reshape(R_TAIL, LANES) for g in gathered]

    in_spec = pl.BlockSpec((RBLK, LANES), lambda i: (i, 0))
    hin_spec = pl.BlockSpec((HBLK, LANES), lambda i: (i, 0))
    rot_spec = pl.BlockSpec((4 * HBLK, LANES), lambda i: (i, 0))
    out_shape = jax.ShapeDtypeStruct((9, R_TOTAL, LANES), jnp.float32)

    planes_head = pl.pallas_call(
        _cov_head_body,
        grid=(H_STEPS,),
        in_specs=[hin_spec] * 3 + [rot_spec],
        out_specs=pl.BlockSpec((9, HBLK, LANES), lambda i: (0, i, 0)),
        out_shape=out_shape,
    )(*shead, rot2)

    planes = pl.pallas_call(
        _cov_tail_body,
        grid=(GRID - N_HEAD,),
        in_specs=[in_spec] * 7 + [pl.BlockSpec(memory_space=pl.ANY)],
        out_specs=pl.BlockSpec((9, RBLK, LANES), lambda i: (0, N_HEAD + i, 0)),
        out_shape=out_shape,
        input_output_aliases={7: 0},
    )(*tail_in, planes_head)

    return planes.reshape(3, 3, M_TOTAL).transpose(2, 0, 1)


# speedup vs baseline: 1.3932x; 1.3932x over previous
import functools

import jax
import jax.numpy as jnp
from jax import lax
from jax.experimental import pallas as pl
from jax.experimental.pallas import tpu as pltpu
from jax.experimental.pallas import tpu_sc as plsc

M_TOTAL = 2097152
B_CLONE = 262144
SIZE = M_TOTAL - B_CLONE

LANES = 128
R_TOTAL = M_TOTAL // LANES
R_HEAD = SIZE // LANES
R_TAIL = B_CLONE // LANES

RBLK = 512
GRID = R_TOTAL // RBLK
N_HEAD = R_HEAD // RBLK


def _cov_math(s0, s1, s2, q0, q1, q2, q3):
    n2 = q0 * q0 + q1 * q1 + q2 * q2 + q3 * q3
    inv = 1.0 / jnp.maximum(jnp.sqrt(n2), 1e-12)
    w, x, y, z = q0 * inv, q1 * inv, q2 * inv, q3 * inv

    e0 = jnp.exp(s0)
    e1 = jnp.exp(s1)
    e2 = jnp.exp(s2)

    m00 = (1.0 - 2.0 * (y * y + z * z)) * e0
    m01 = (2.0 * (x * y - w * z)) * e1
    m02 = (2.0 * (x * z + w * y)) * e2
    m10 = (2.0 * (x * y + w * z)) * e0
    m11 = (1.0 - 2.0 * (x * x + z * z)) * e1
    m12 = (2.0 * (y * z - w * x)) * e2
    m20 = (2.0 * (x * z - w * y)) * e0
    m21 = (2.0 * (y * z + w * x)) * e1
    m22 = (1.0 - 2.0 * (x * x + y * y)) * e2

    c00 = m00 * m00 + m01 * m01 + m02 * m02
    c01 = m00 * m10 + m01 * m11 + m02 * m12
    c02 = m00 * m20 + m01 * m21 + m02 * m22
    c11 = m10 * m10 + m11 * m11 + m12 * m12
    c12 = m10 * m20 + m11 * m21 + m12 * m22
    c22 = m20 * m20 + m21 * m21 + m22 * m22
    return c00, c01, c02, c11, c12, c22


def _store_planes(outr, c00, c01, c02, c11, c12, c22):
    outr[0] = c00
    outr[1] = c01
    outr[2] = c02
    outr[3] = c01
    outr[4] = c11
    outr[5] = c12
    outr[6] = c02
    outr[7] = c12
    outr[8] = c22


def _cov_body(s0r, s1r, s2r, q0r, q1r, q2r, q3r, outr):
    _store_planes(outr, *_cov_math(s0r[...], s1r[...], s2r[...],
                                   q0r[...], q1r[...], q2r[...], q3r[...]))


def _cov_tail_body(s0r, s1r, s2r, q0r, q1r, q2r, q3r, alias_r, outr):
    del alias_r
    _store_planes(outr, *_cov_math(s0r[...], s1r[...], s2r[...],
                                   q0r[...], q1r[...], q2r[...], q3r[...]))


def _sc_gather(tables, idx):
    nt = len(tables)
    info = plsc.get_sparse_core_info()
    nc, ns = info.num_cores, info.num_subcores
    nw = nc * ns
    b_per_w = B_CLONE // nw
    mesh = plsc.VectorSubcoreMesh(core_axis_name="c", subcore_axis_name="s")

    @functools.partial(
        pl.kernel,
        mesh=mesh,
        out_type=[jax.ShapeDtypeStruct((B_CLONE,), jnp.float32)
                  for _ in range(nt)],
        scratch_types=(
            [pltpu.VMEM((b_per_w,), jnp.int32)]
            + [pltpu.VMEM((b_per_w,), jnp.float32) for _ in range(nt)]
            + [pltpu.SemaphoreType.DMA for _ in range(nt)]
        ),
    )
    def gather_k(*refs):
        tbls = refs[:nt]
        idx_hbm = refs[nt]
        outs = refs[nt + 1:2 * nt + 1]
        idx_v = refs[2 * nt + 1]
        stages = refs[2 * nt + 2:3 * nt + 2]
        sems = refs[3 * nt + 2:]
        wid = lax.axis_index("s") * nc + lax.axis_index("c")
        base = wid * b_per_w

        pltpu.sync_copy(idx_hbm.at[pl.ds(base, b_per_w)], idx_v)
        copies = [
            pltpu.async_copy(tbls[t].at[idx_v], stages[t], sems[t])
            for t in range(nt)
        ]
        for t in range(nt):
            copies[t].wait()
            pltpu.sync_copy(stages[t], outs[t].at[pl.ds(base, b_per_w)])

    return gather_k(*tables, idx)


def kernel(scale, rotation, idx):
    comps = [scale[:, c] for c in range(3)] + [rotation[:, c] for c in range(4)]
    gathered = _sc_gather([c.reshape(M_TOTAL) for c in comps], idx)

    head_in = [c.reshape(R_TOTAL, LANES) for c in comps]
    tail_in = [g.reshape(R_TAIL, LANES) for g in gathered]

    in_spec = pl.BlockSpec((RBLK, LANES), lambda i: (i, 0))

    planes_head = pl.pallas_call(
        _cov_body,
        grid=(N_HEAD,),
        in_specs=[in_spec] * 7,
        out_specs=pl.BlockSpec((9, RBLK, LANES), lambda i: (0, i, 0)),
        out_shape=jax.ShapeDtypeStruct((9, R_TOTAL, LANES), jnp.float32),
    )(*head_in)

    planes = pl.pallas_call(
        _cov_tail_body,
        grid=(GRID - N_HEAD,),
        in_specs=[in_spec] * 7 + [pl.BlockSpec(memory_space=pl.ANY)],
        out_specs=pl.BlockSpec((9, RBLK, LANES), lambda i: (0, N_HEAD + i, 0)),
        out_shape=jax.ShapeDtypeStruct((9, R_TOTAL, LANES), jnp.float32),
        input_output_aliases={7: 0},
    )(*tail_in, planes_head)

    return planes.reshape(3, 3, M_TOTAL).transpose(2, 0, 1)


# speedup vs baseline: 1.4481x; 1.0394x over previous
import functools

import jax
import jax.numpy as jnp
from jax import lax
from jax.experimental import pallas as pl
from jax.experimental.pallas import tpu as pltpu
from jax.experimental.pallas import tpu_sc as plsc

M_TOTAL = 2097152
B_CLONE = 262144
SIZE = M_TOTAL - B_CLONE

LANES = 128
R_TOTAL = M_TOTAL // LANES
R_HEAD = SIZE // LANES
R_TAIL = B_CLONE // LANES

RBLK = 512
GRID = R_TOTAL // RBLK
N_HEAD = R_HEAD // RBLK


def _cov_math(s0, s1, s2, q0, q1, q2, q3):
    n2 = q0 * q0 + q1 * q1 + q2 * q2 + q3 * q3
    inv = 1.0 / jnp.maximum(jnp.sqrt(n2), 1e-12)
    w, x, y, z = q0 * inv, q1 * inv, q2 * inv, q3 * inv

    e0 = jnp.exp(s0)
    e1 = jnp.exp(s1)
    e2 = jnp.exp(s2)

    m00 = (1.0 - 2.0 * (y * y + z * z)) * e0
    m01 = (2.0 * (x * y - w * z)) * e1
    m02 = (2.0 * (x * z + w * y)) * e2
    m10 = (2.0 * (x * y + w * z)) * e0
    m11 = (1.0 - 2.0 * (x * x + z * z)) * e1
    m12 = (2.0 * (y * z - w * x)) * e2
    m20 = (2.0 * (x * z - w * y)) * e0
    m21 = (2.0 * (y * z + w * x)) * e1
    m22 = (1.0 - 2.0 * (x * x + y * y)) * e2

    c00 = m00 * m00 + m01 * m01 + m02 * m02
    c01 = m00 * m10 + m01 * m11 + m02 * m12
    c02 = m00 * m20 + m01 * m21 + m02 * m22
    c11 = m10 * m10 + m11 * m11 + m12 * m12
    c12 = m10 * m20 + m11 * m21 + m12 * m22
    c22 = m20 * m20 + m21 * m21 + m22 * m22
    return c00, c01, c02, c11, c12, c22


def _store_planes(outr, c00, c01, c02, c11, c12, c22):
    outr[0] = c00
    outr[1] = c01
    outr[2] = c02
    outr[3] = c01
    outr[4] = c11
    outr[5] = c12
    outr[6] = c02
    outr[7] = c12
    outr[8] = c22


def _cov_body(s0r, s1r, s2r, q0r, q1r, q2r, q3r, outr):
    _store_planes(outr, *_cov_math(s0r[...], s1r[...], s2r[...],
                                   q0r[...], q1r[...], q2r[...], q3r[...]))


def _cov_tail_body(s0r, s1r, s2r, q0r, q1r, q2r, q3r, alias_r, outr):
    del alias_r
    _store_planes(outr, *_cov_math(s0r[...], s1r[...], s2r[...],
                                   q0r[...], q1r[...], q2r[...], q3r[...]))


def _sc_gather(tables, idx):
    nt = len(tables)
    info = plsc.get_sparse_core_info()
    nc, ns = info.num_cores, info.num_subcores
    nw = nc * ns
    b_per_w = B_CLONE // nw
    mesh = plsc.VectorSubcoreMesh(core_axis_name="c", subcore_axis_name="s")

    @functools.partial(
        pl.kernel,
        mesh=mesh,
        out_type=[jax.ShapeDtypeStruct((B_CLONE,), jnp.float32)
                  for _ in range(nt)],
        scratch_types=(
            [pltpu.VMEM((b_per_w,), jnp.int32)]
            + [pltpu.VMEM((b_per_w,), jnp.float32) for _ in range(nt)]
            + [pltpu.SemaphoreType.DMA for _ in range(nt)]
        ),
    )
    def gather_k(*refs):
        tbls = refs[:nt]
        idx_hbm = refs[nt]
        outs = refs[nt + 1:2 * nt + 1]
        idx_v = refs[2 * nt + 1]
        stages = refs[2 * nt + 2:3 * nt + 2]
        sems = refs[3 * nt + 2:]
        wid = lax.axis_index("s") * nc + lax.axis_index("c")
        base = wid * b_per_w

        pltpu.sync_copy(idx_hbm.at[pl.ds(base, b_per_w)], idx_v)
        copies = [
            pltpu.async_copy(tbls[t].at[idx_v], stages[t], sems[t])
            for t in range(nt)
        ]
        for t in range(nt):
            copies[t].wait()
            pltpu.sync_copy(stages[t], outs[t].at[pl.ds(base, b_per_w)])

    return gather_k(*tables, idx)


def kernel(scale, rotation, idx):
    scomps = [scale[:, c] for c in range(3)]
    rcomps = [rotation[:, c] for c in range(4)]
    comps = scomps + rcomps
    gathered_s = _sc_gather([c.reshape(M_TOTAL) for c in scomps], idx)
    gathered_r = _sc_gather([c.reshape(M_TOTAL) for c in rcomps], idx)
    gathered = list(gathered_s) + list(gathered_r)

    head_in = [c.reshape(R_TOTAL, LANES) for c in comps]
    tail_in = [g.reshape(R_TAIL, LANES) for g in gathered]

    in_spec = pl.BlockSpec((RBLK, LANES), lambda i: (i, 0))

    planes_head = pl.pallas_call(
        _cov_body,
        grid=(N_HEAD,),
        in_specs=[in_spec] * 7,
        out_specs=pl.BlockSpec((9, RBLK, LANES), lambda i: (0, i, 0)),
        out_shape=jax.ShapeDtypeStruct((9, R_TOTAL, LANES), jnp.float32),
    )(*head_in)

    planes = pl.pallas_call(
        _cov_tail_body,
        grid=(GRID - N_HEAD,),
        in_specs=[in_spec] * 7 + [pl.BlockSpec(memory_space=pl.ANY)],
        out_specs=pl.BlockSpec((9, RBLK, LANES), lambda i: (0, N_HEAD + i, 0)),
        out_shape=jax.ShapeDtypeStruct((9, R_TOTAL, LANES), jnp.float32),
        input_output_aliases={7: 0},
    )(*tail_in, planes_head)

    return planes.reshape(3, 3, M_TOTAL).transpose(2, 0, 1)
